# baseline (speedup 1.0000x reference)
import jax
import jax.numpy as jnp
from jax.experimental import pallas as pl
from jax.experimental.pallas import tpu as pltpu

B, N, CDIM = 8, 512, 3
K = 10
D = 128
POOL = 4
FC1, FC2 = 512, 128

_LOG2E = 1.4426950408889634


def _normalize_prev(l, b, h_scr, st_scr, gall_ref, beall_ref):
    slot = (l - 1) % 2
    full = st_scr[...]
    st2 = jnp.where(slot == 0, full[0], full[1])
    inv_n = 1.0 / float(B * N)
    mean = st2[0] * inv_n
    var = st2[1] * inv_n - mean * mean
    rstd = jax.lax.rsqrt(var + 1e-5)
    g = gall_ref[l - 1]
    be = beall_ref[l - 1]
    scale = g * rstd
    shift = be - mean * scale
    acc = None
    for kk in range(K):
        hk = h_scr[b * K + kk].astype(jnp.float32)
        hn = hk * scale[kk:kk + 1, :] + shift[kk:kk + 1, :]
        hs = hn * pl.reciprocal(1.0 + jnp.abs(hn), approx=True)
        acc = hs if acc is None else acc + hs
    return (acc * (1.0 / K)).astype(jnp.bfloat16)


def _gcn_kernel(c_ref, ct_ref, v_ref, wall_ref, ball_ref, gall_ref,
                beall_ref, mu_ref, nv_ref, o_ref,
                a_scr, h_scr, vx_scr, dm_scr, st_scr):
    l = pl.program_id(0)
    b = pl.program_id(1)
    k = pl.program_id(2)

    @pl.when(jnp.logical_and(l == 0, k == 0))
    def _():
        cb = c_ref[0]
        ct = ct_ref[0]
        acc = None
        for cc in range(CDIM):
            dif = cb[:, cc:cc + 1] - ct[cc:cc + 1, :]
            sq = dif * dif
            acc = sq if acc is None else acc + sq
        dm_scr[...] = jnp.sqrt(acc + 1e-12)
        vx_scr[b, :, 0:D] = v_ref[0]
        vx_scr[b, :, D:2 * D] = jnp.ones((N, D), jnp.bfloat16)

    @pl.when(l == 0)
    def _():
        d0 = dm_scr[...] - mu_ref[k]
        e = jax.lax.exp2(d0 * d0 * nv_ref[k])
        a_scr[b * K + k] = e.astype(jnp.bfloat16)

    @pl.when(jnp.logical_and(jnp.logical_and(l >= 1, l <= 2), k == 0))
    def _():
        vx_scr[b, :, 0:D] = _normalize_prev(l, b, h_scr, st_scr,
                                            gall_ref, beall_ref)

    @pl.when(jnp.logical_and(l == 3, k == 0))
    def _():
        o_ref[0] = _normalize_prev(l, b, h_scr, st_scr,
                                   gall_ref, beall_ref)

    @pl.when(l <= 2)
    def _():
        lw = jnp.minimum(l, 2)
        a = a_scr[b * K + k]
        maug = jnp.dot(a, vx_scr[b],
                       preferred_element_type=jnp.float32)
        rs = maug[:, D:D + 1]
        m = maug[:, 0:D] * pl.reciprocal(rs + 1e-9, approx=True)
        h = jnp.dot(m.astype(jnp.bfloat16), wall_ref[lw, k],
                    preferred_element_type=jnp.float32) + ball_ref[lw, k]
        h_scr[b * K + k] = h.astype(jnp.bfloat16)

        s1 = jnp.sum(h, axis=0, keepdims=True)
        s2 = jnp.sum(h * h, axis=0, keepdims=True)
        upd = jnp.stack([jnp.broadcast_to(s1, (K, D)),
                         jnp.broadcast_to(s2, (K, D))], axis=0)
        selk = jax.lax.broadcasted_iota(jnp.int32, (2, K, D), 1) == k
        contrib = jnp.where(selk, upd, 0.0)
        slot = l % 2
        slotmask = (jax.lax.broadcasted_iota(jnp.int32, (2, 2, K, D), 0)
                    == slot)
        first = jnp.logical_and(b == 0, k == 0)
        full = st_scr[...]
        cleared = jnp.where(jnp.logical_and(first, slotmask), 0.0, full)
        st_scr[...] = cleared + jnp.where(slotmask, contrib[None], 0.0)


def _gcn_call(C, CT, Vb, Wall, ball, gall, beall, mu, nv2):
    return pl.pallas_call(
        _gcn_kernel,
        grid=(4, B, K),
        in_specs=[
            pl.BlockSpec((1, N, CDIM), lambda l, b, k: (b, 0, 0)),
            pl.BlockSpec((1, CDIM, N), lambda l, b, k: (b, 0, 0)),
            pl.BlockSpec((1, N, D), lambda l, b, k: (b, 0, 0)),
            pl.BlockSpec((3, K, D, D), lambda l, b, k: (0, 0, 0, 0)),
            pl.BlockSpec((3, K, 1, D), lambda l, b, k: (0, 0, 0, 0)),
            pl.BlockSpec((3, K, D), lambda l, b, k: (0, 0, 0)),
            pl.BlockSpec((3, K, D), lambda l, b, k: (0, 0, 0)),
            pl.BlockSpec(memory_space=pltpu.SMEM),
            pl.BlockSpec(memory_space=pltpu.SMEM),
        ],
        out_specs=pl.BlockSpec((1, N, D), lambda l, b, k: (b, 0, 0)),
        out_shape=jax.ShapeDtypeStruct((B, N, D), jnp.bfloat16),
        scratch_shapes=[
            pltpu.VMEM((B * K, N, N), jnp.bfloat16),
            pltpu.VMEM((B * K, N, D), jnp.bfloat16),
            pltpu.VMEM((B, N, 2 * D), jnp.bfloat16),
            pltpu.VMEM((N, N), jnp.float32),
            pltpu.VMEM((2, 2, K, D), jnp.float32),
        ],
        compiler_params=pltpu.CompilerParams(
            dimension_semantics=("arbitrary", "arbitrary", "arbitrary")),
    )(C, CT, Vb, Wall, ball, gall, beall, mu, nv2)


def _head_kernel(vx_ref, w1_ref, bf1_ref, w2_ref, bf2_ref, o_ref, acc_ref):
    j = pl.program_id(0)
    nj = pl.num_programs(0)
    d = vx_ref.shape[-1]
    rows = w1_ref.shape[0]

    @pl.when(j == 0)
    def _():
        acc_ref[...] = jnp.zeros_like(acc_ref)

    v = vx_ref[...]
    p = v.reshape(B, rows, POOL, d).max(axis=2)
    part = None
    for i in range(rows):
        t = jnp.dot(p[:, i, :], w1_ref[i],
                    preferred_element_type=jnp.float32)
        part = t if part is None else part + t
    acc_ref[...] += part

    @pl.when(j == nj - 1)
    def _():
        h1 = jnp.maximum(acc_ref[...] + bf1_ref[...], 0.0)
        o = jnp.dot(h1.astype(jnp.bfloat16), w2_ref[...],
                    preferred_element_type=jnp.float32) + bf2_ref[...]
        o_ref[...] = jnp.maximum(o, 0.0)


def _head_call(Vx, W1r, bf1, W2, bf2):
    d = Vx.shape[-1]
    n2 = N // POOL
    rows = 16
    nsteps = n2 // rows
    return pl.pallas_call(
        _head_kernel,
        grid=(nsteps,),
        in_specs=[
            pl.BlockSpec((B, rows * POOL, d), lambda j: (0, j, 0)),
            pl.BlockSpec((rows, d, FC1), lambda j: (j, 0, 0)),
            pl.BlockSpec((1, FC1), lambda j: (0, 0)),
            pl.BlockSpec((FC1, FC2), lambda j: (0, 0)),
            pl.BlockSpec((1, FC2), lambda j: (0, 0)),
        ],
        out_specs=pl.BlockSpec((B, FC2), lambda j: (0, 0)),
        out_shape=jax.ShapeDtypeStruct((B, FC2), jnp.float32),
        scratch_shapes=[pltpu.VMEM((B, FC1), jnp.float32)],
        compiler_params=pltpu.CompilerParams(
            dimension_semantics=("arbitrary",)),
    )(Vx, W1r, bf1, W2, bf2)


def kernel(V, C, mu, sigma, W1, b1, g1, be1, W2, b2, g2, be2,
           W3, b3, g3, be3, Wf1, bf1, Wf2, bf2):
    CT = jnp.swapaxes(C, 1, 2)
    nv2 = -_LOG2E / (2.0 * sigma * sigma + 1e-6)

    Vb = jnp.pad(V, ((0, 0), (0, 0), (0, D - V.shape[-1]))).astype(
        jnp.bfloat16)
    W1p = jnp.pad(W1, ((0, 0), (0, D - W1.shape[1]), (0, 0)))
    Wall = jnp.stack([W1p, W2, W3]).astype(jnp.bfloat16)
    ball = jnp.stack([b1, b2, b3])[:, :, None, :]
    gall = jnp.stack([g1, g2, g3])
    beall = jnp.stack([be1, be2, be3])

    Vx = _gcn_call(C, CT, Vb, Wall, ball, gall, beall, mu, nv2)

    W1r = Wf1.reshape(N // POOL, D, FC1).astype(jnp.bfloat16)
    out = _head_call(Vx, W1r, bf1[None, :], Wf2.astype(jnp.bfloat16),
                     bf2[None, :])
    return out


# speedup vs baseline: 1.7247x; 1.7247x over previous
import jax
import jax.numpy as jnp
from jax.experimental import pallas as pl
from jax.experimental.pallas import tpu as pltpu

B, N, CDIM = 8, 512, 3
K = 10
D = 128
POOL = 4
FC1, FC2 = 512, 128

_LOG2E = 1.4426950408889634


def _normalize_prev(l, b, h_scr, st_scr, gall_ref, beall_ref):
    slot = (l - 1) % 2
    full = st_scr[...]
    st2 = jnp.where(slot == 0, full[0], full[1])
    inv_n = 1.0 / float(B * N)
    mean = st2[0] * inv_n
    var = st2[1] * inv_n - mean * mean
    rstd = jax.lax.rsqrt(var + 1e-5)
    g = gall_ref[l - 1]
    be = beall_ref[l - 1]
    scale = (g * rstd).astype(jnp.bfloat16)
    shift = (be - mean * g * rstd).astype(jnp.bfloat16)
    acc = None
    for kk in range(K):
        hk = h_scr[b * K + kk]
        hn = (hk * scale[kk:kk + 1, :]
              + shift[kk:kk + 1, :]).astype(jnp.float32)
        hs = hn * pl.reciprocal(1.0 + jnp.abs(hn), approx=True)
        acc = hs if acc is None else acc + hs
    return (acc * (1.0 / K)).astype(jnp.bfloat16)


KG = 5
NJ = K // KG


def _gcn_kernel(c_ref, ct_ref, v_ref, wall_ref, ball_ref, gall_ref,
                beall_ref, mu_ref, nv_ref, o_ref,
                a_scr, h_scr, vx_scr, dm_scr, st_scr):
    l = pl.program_id(0)
    b = pl.program_id(1)
    j = pl.program_id(2)

    @pl.when(jnp.logical_and(l == 0, j == 0))
    def _():
        cb = c_ref[0]
        ct = ct_ref[0]
        acc = None
        for cc in range(CDIM):
            dif = cb[:, cc:cc + 1] - ct[cc:cc + 1, :]
            sq = dif * dif
            acc = sq if acc is None else acc + sq
        dm_scr[...] = jnp.sqrt(acc + 1e-12)
        vx_scr[b, :, 0:D] = v_ref[0]
        vx_scr[b, :, D:2 * D] = jnp.ones((N, D), jnp.bfloat16)

    @pl.when(l == 0)
    def _():
        dm = dm_scr[...]
        for kk in range(KG):
            k = j * KG + kk
            d0 = dm - mu_ref[k]
            e = jax.lax.exp2(d0 * d0 * nv_ref[k])
            a_scr[b * K + k] = e.astype(jnp.bfloat16)

    @pl.when(jnp.logical_and(jnp.logical_and(l >= 1, l <= 2), j == 0))
    def _():
        vx_scr[b, :, 0:D] = _normalize_prev(l, b, h_scr, st_scr,
                                            gall_ref, beall_ref)

    @pl.when(jnp.logical_and(l == 3, j == 0))
    def _():
        o_ref[0] = _normalize_prev(l, b, h_scr, st_scr,
                                   gall_ref, beall_ref)

    @pl.when(l <= 2)
    def _():
        lw = jnp.minimum(l, 2)
        ag = a_scr[pl.ds(b * K + j * KG, KG)]
        agr = ag.reshape(KG * N, N)
        maug = jnp.dot(agr, vx_scr[b],
                       preferred_element_type=jnp.float32)
        rs = maug[:, D:D + 1]
        m = maug[:, 0:D] * pl.reciprocal(rs + 1e-9, approx=True)
        m_bf = m.astype(jnp.bfloat16)
        s1l, s2l = [], []
        for kk in range(KG):
            k = j * KG + kk
            h = jnp.dot(m_bf[kk * N:(kk + 1) * N], wall_ref[lw, k],
                        preferred_element_type=jnp.float32) \
                + ball_ref[lw, k]
            h_scr[b * K + k] = h.astype(jnp.bfloat16)
            s1l.append(jnp.sum(h, axis=0, keepdims=True))
            s2l.append(jnp.sum(h * h, axis=0, keepdims=True))
        s1g = jnp.concatenate(s1l, axis=0)
        s2g = jnp.concatenate(s2l, axis=0)
        z = jnp.zeros((K - KG, D), jnp.float32)
        s1k = jnp.where(j == 0, jnp.concatenate([s1g, z]),
                        jnp.concatenate([z, s1g]))
        s2k = jnp.where(j == 0, jnp.concatenate([s2g, z]),
                        jnp.concatenate([z, s2g]))
        contrib = jnp.stack([s1k, s2k], axis=0)
        slot = l % 2
        slotmask = (jax.lax.broadcasted_iota(jnp.int32, (2, 2, K, D), 0)
                    == slot)
        first = jnp.logical_and(b == 0, j == 0)
        full = st_scr[...]
        cleared = jnp.where(jnp.logical_and(first, slotmask), 0.0, full)
        st_scr[...] = cleared + jnp.where(slotmask, contrib[None], 0.0)


def _gcn_call(C, CT, Vb, Wall, ball, gall, beall, mu, nv2):
    return pl.pallas_call(
        _gcn_kernel,
        grid=(4, B, NJ),
        in_specs=[
            pl.BlockSpec((1, N, CDIM), lambda l, b, k: (b, 0, 0)),
            pl.BlockSpec((1, CDIM, N), lambda l, b, k: (b, 0, 0)),
            pl.BlockSpec((1, N, D), lambda l, b, k: (b, 0, 0)),
            pl.BlockSpec((3, K, D, D), lambda l, b, k: (0, 0, 0, 0)),
            pl.BlockSpec((3, K, 1, D), lambda l, b, k: (0, 0, 0, 0)),
            pl.BlockSpec((3, K, D), lambda l, b, k: (0, 0, 0)),
            pl.BlockSpec((3, K, D), lambda l, b, k: (0, 0, 0)),
            pl.BlockSpec(memory_space=pltpu.SMEM),
            pl.BlockSpec(memory_space=pltpu.SMEM),
        ],
        out_specs=pl.BlockSpec((1, N, D), lambda l, b, k: (b, 0, 0)),
        out_shape=jax.ShapeDtypeStruct((B, N, D), jnp.bfloat16),
        scratch_shapes=[
            pltpu.VMEM((B * K, N, N), jnp.bfloat16),
            pltpu.VMEM((B * K, N, D), jnp.bfloat16),
            pltpu.VMEM((B, N, 2 * D), jnp.bfloat16),
            pltpu.VMEM((N, N), jnp.float32),
            pltpu.VMEM((2, 2, K, D), jnp.float32),
        ],
        compiler_params=pltpu.CompilerParams(
            dimension_semantics=("arbitrary", "arbitrary", "arbitrary")),
    )(C, CT, Vb, Wall, ball, gall, beall, mu, nv2)


def _head_kernel(vx_ref, w1_ref, bf1_ref, w2_ref, bf2_ref, o_ref, acc_ref):
    j = pl.program_id(0)
    nj = pl.num_programs(0)
    d = vx_ref.shape[-1]
    rows = w1_ref.shape[0]

    @pl.when(j == 0)
    def _():
        acc_ref[...] = jnp.zeros_like(acc_ref)

    v = vx_ref[...]
    p = v.reshape(B, rows, POOL, d).max(axis=2)
    part = None
    for i in range(rows):
        t = jnp.dot(p[:, i, :], w1_ref[i],
                    preferred_element_type=jnp.float32)
        part = t if part is None else part + t
    acc_ref[...] += part

    @pl.when(j == nj - 1)
    def _():
        h1 = jnp.maximum(acc_ref[...] + bf1_ref[...], 0.0)
        o = jnp.dot(h1.astype(jnp.bfloat16), w2_ref[...],
                    preferred_element_type=jnp.float32) + bf2_ref[...]
        o_ref[...] = jnp.maximum(o, 0.0)


def _head_call(Vx, W1r, bf1, W2, bf2):
    d = Vx.shape[-1]
    n2 = N // POOL
    rows = 16
    nsteps = n2 // rows
    return pl.pallas_call(
        _head_kernel,
        grid=(nsteps,),
        in_specs=[
            pl.BlockSpec((B, rows * POOL, d), lambda j: (0, j, 0)),
            pl.BlockSpec((rows, d, FC1), lambda j: (j, 0, 0)),
            pl.BlockSpec((1, FC1), lambda j: (0, 0)),
            pl.BlockSpec((FC1, FC2), lambda j: (0, 0)),
            pl.BlockSpec((1, FC2), lambda j: (0, 0)),
        ],
        out_specs=pl.BlockSpec((B, FC2), lambda j: (0, 0)),
        out_shape=jax.ShapeDtypeStruct((B, FC2), jnp.float32),
        scratch_shapes=[pltpu.VMEM((B, FC1), jnp.float32)],
        compiler_params=pltpu.CompilerParams(
            dimension_semantics=("arbitrary",)),
    )(Vx, W1r, bf1, W2, bf2)


def kernel(V, C, mu, sigma, W1, b1, g1, be1, W2, b2, g2, be2,
           W3, b3, g3, be3, Wf1, bf1, Wf2, bf2):
    CT = jnp.swapaxes(C, 1, 2)
    nv2 = -_LOG2E / (2.0 * sigma * sigma + 1e-6)

    Vb = jnp.pad(V, ((0, 0), (0, 0), (0, D - V.shape[-1]))).astype(
        jnp.bfloat16)
    W1p = jnp.pad(W1, ((0, 0), (0, D - W1.shape[1]), (0, 0)))
    Wall = jnp.stack([W1p, W2, W3]).astype(jnp.bfloat16)
    ball = jnp.stack([b1, b2, b3])[:, :, None, :]
    gall = jnp.stack([g1, g2, g3])
    beall = jnp.stack([be1, be2, be3])

    Vx = _gcn_call(C, CT, Vb, Wall, ball, gall, beall, mu, nv2)

    W1r = Wf1.reshape(N // POOL, D, FC1).astype(jnp.bfloat16)
    out = _head_call(Vx, W1r, bf1[None, :], Wf2.astype(jnp.bfloat16),
                     bf2[None, :])
    return out


# speedup vs baseline: 1.9133x; 1.1093x over previous
import jax
import jax.numpy as jnp
from jax.experimental import pallas as pl
from jax.experimental.pallas import tpu as pltpu

B, N, CDIM = 8, 512, 3
K = 10
D = 128
POOL = 4
FC1, FC2 = 512, 128

_LOG2E = 1.4426950408889634


def _normalize_prev(l, b, h_scr, st_scr, gall_ref, beall_ref):
    slot = (l - 1) % 2
    full = st_scr[...]
    st2 = jnp.where(slot == 0, full[0], full[1])
    inv_n = 1.0 / float(B * N)
    mean = st2[0] * inv_n
    var = st2[1] * inv_n - mean * mean
    rstd = jax.lax.rsqrt(var + 1e-5)
    g = gall_ref[l - 1]
    be = beall_ref[l - 1]
    scale = (g * rstd).astype(jnp.bfloat16)
    shift = (be - mean * g * rstd).astype(jnp.bfloat16)
    acc = None
    for kk in range(K):
        hk = h_scr[b * K + kk]
        hn = (hk * scale[kk:kk + 1, :]
              + shift[kk:kk + 1, :]).astype(jnp.float32)
        hs = hn * pl.reciprocal(1.0 + jnp.abs(hn), approx=True)
        acc = hs if acc is None else acc + hs
    return (acc * (1.0 / K)).astype(jnp.bfloat16)


KG = 5
NJ = K // KG


def _gcn_kernel(c_ref, ct_ref, v_ref, wall_ref, ball_ref, gall_ref,
                beall_ref, mu_ref, nv_ref, o_ref,
                a_scr, h_scr, vx_scr, dm_scr, st_scr):
    l = pl.program_id(0)
    b = pl.program_id(1)
    j = pl.program_id(2)

    @pl.when(jnp.logical_and(l == 0, j == 0))
    def _():
        cb = c_ref[0]
        ct = ct_ref[0]
        acc = None
        for cc in range(CDIM):
            dif = cb[:, cc:cc + 1] - ct[cc:cc + 1, :]
            sq = dif * dif
            acc = sq if acc is None else acc + sq
        dm_scr[...] = jnp.sqrt(acc + 1e-12)
        vx_scr[b, :, 0:D] = v_ref[0]
        vx_scr[b, :, D:2 * D] = jnp.ones((N, D), jnp.bfloat16)

    @pl.when(l == 0)
    def _():
        dm = dm_scr[...]
        for kk in range(KG):
            k = j * KG + kk
            d0 = dm - mu_ref[k]
            e = jax.lax.exp2(d0 * d0 * nv_ref[k])
            a_scr[b * K + k] = e.astype(jnp.bfloat16)

    @pl.when(jnp.logical_and(jnp.logical_and(l >= 1, l <= 2), j == 0))
    def _():
        vx_scr[b, :, 0:D] = _normalize_prev(l, b, h_scr, st_scr,
                                            gall_ref, beall_ref)

    @pl.when(jnp.logical_and(l == 3, j == 0))
    def _():
        o_ref[0] = _normalize_prev(l, b, h_scr, st_scr,
                                   gall_ref, beall_ref)

    @pl.when(l <= 2)
    def _():
        lw = jnp.minimum(l, 2)
        ag = a_scr[pl.ds(b * K + j * KG, KG)]
        agr = ag.reshape(KG * N, N)
        maug = jnp.dot(agr, vx_scr[b],
                       preferred_element_type=jnp.float32)
        rs = maug[:, D:D + 1]
        m = maug[:, 0:D] * pl.reciprocal(rs + 1e-9, approx=True)
        m_bf = m.astype(jnp.bfloat16)
        s1l, s2l = [], []
        for kk in range(KG):
            k = j * KG + kk
            h = jnp.dot(m_bf[kk * N:(kk + 1) * N], wall_ref[lw, k],
                        preferred_element_type=jnp.float32) \
                + ball_ref[lw, k]
            h_scr[b * K + k] = h.astype(jnp.bfloat16)
            s1l.append(jnp.sum(h, axis=0, keepdims=True))
            s2l.append(jnp.sum(h * h, axis=0, keepdims=True))
        s1g = jnp.concatenate(s1l, axis=0)
        s2g = jnp.concatenate(s2l, axis=0)
        z = jnp.zeros((K - KG, D), jnp.float32)
        s1k = jnp.where(j == 0, jnp.concatenate([s1g, z]),
                        jnp.concatenate([z, s1g]))
        s2k = jnp.where(j == 0, jnp.concatenate([s2g, z]),
                        jnp.concatenate([z, s2g]))
        contrib = jnp.stack([s1k, s2k], axis=0)
        slot = l % 2
        slotmask = (jax.lax.broadcasted_iota(jnp.int32, (2, 2, K, D), 0)
                    == slot)
        first = jnp.logical_and(b == 0, j == 0)
        full = st_scr[...]
        cleared = jnp.where(jnp.logical_and(first, slotmask), 0.0, full)
        st_scr[...] = cleared + jnp.where(slotmask, contrib[None], 0.0)


def _gcn_call(C, CT, Vb, Wall, ball, gall, beall, mu, nv2):
    return pl.pallas_call(
        _gcn_kernel,
        grid=(4, B, NJ),
        in_specs=[
            pl.BlockSpec((1, N, CDIM), lambda l, b, k: (b, 0, 0)),
            pl.BlockSpec((1, CDIM, N), lambda l, b, k: (b, 0, 0)),
            pl.BlockSpec((1, N, D), lambda l, b, k: (b, 0, 0)),
            pl.BlockSpec((3, K, D, D), lambda l, b, k: (0, 0, 0, 0)),
            pl.BlockSpec((3, K, 1, D), lambda l, b, k: (0, 0, 0, 0)),
            pl.BlockSpec((3, K, D), lambda l, b, k: (0, 0, 0)),
            pl.BlockSpec((3, K, D), lambda l, b, k: (0, 0, 0)),
            pl.BlockSpec(memory_space=pltpu.SMEM),
            pl.BlockSpec(memory_space=pltpu.SMEM),
        ],
        out_specs=pl.BlockSpec((1, N, D), lambda l, b, k: (b, 0, 0)),
        out_shape=jax.ShapeDtypeStruct((B, N, D), jnp.bfloat16),
        scratch_shapes=[
            pltpu.VMEM((B * K, N, N), jnp.bfloat16),
            pltpu.VMEM((B * K, N, D), jnp.bfloat16),
            pltpu.VMEM((B, N, 2 * D), jnp.bfloat16),
            pltpu.VMEM((N, N), jnp.float32),
            pltpu.VMEM((2, 2, K, D), jnp.float32),
        ],
        compiler_params=pltpu.CompilerParams(
            dimension_semantics=("arbitrary", "arbitrary", "arbitrary")),
    )(C, CT, Vb, Wall, ball, gall, beall, mu, nv2)


def _head_kernel(vx_ref, w1_ref, bf1_ref, w2_ref, bf2_ref, o_ref, acc_ref):
    j = pl.program_id(0)
    nj = pl.num_programs(0)
    d = vx_ref.shape[-1]
    rows = w1_ref.shape[0]

    @pl.when(j == 0)
    def _():
        acc_ref[...] = jnp.zeros_like(acc_ref)

    v = vx_ref[...]
    p = v.reshape(B, rows, POOL, d).max(axis=2)
    part = None
    for i in range(rows):
        t = jnp.dot(p[:, i, :], w1_ref[i].astype(jnp.bfloat16),
                    preferred_element_type=jnp.float32)
        part = t if part is None else part + t
    acc_ref[...] += part

    @pl.when(j == nj - 1)
    def _():
        h1 = jnp.maximum(acc_ref[...] + bf1_ref[...], 0.0)
        o = jnp.dot(h1.astype(jnp.bfloat16),
                    w2_ref[...].astype(jnp.bfloat16),
                    preferred_element_type=jnp.float32) + bf2_ref[...]
        o_ref[...] = jnp.maximum(o, 0.0)


def _head_call(Vx, W1r, bf1, W2, bf2):
    d = Vx.shape[-1]
    n2 = N // POOL
    rows = 16
    nsteps = n2 // rows
    return pl.pallas_call(
        _head_kernel,
        grid=(nsteps,),
        in_specs=[
            pl.BlockSpec((B, rows * POOL, d), lambda j: (0, j, 0)),
            pl.BlockSpec((rows, d, FC1), lambda j: (j, 0, 0)),
            pl.BlockSpec((1, FC1), lambda j: (0, 0)),
            pl.BlockSpec((FC1, FC2), lambda j: (0, 0)),
            pl.BlockSpec((1, FC2), lambda j: (0, 0)),
        ],
        out_specs=pl.BlockSpec((B, FC2), lambda j: (0, 0)),
        out_shape=jax.ShapeDtypeStruct((B, FC2), jnp.float32),
        scratch_shapes=[pltpu.VMEM((B, FC1), jnp.float32)],
        compiler_params=pltpu.CompilerParams(
            dimension_semantics=("arbitrary",)),
    )(Vx, W1r, bf1, W2, bf2)


def kernel(V, C, mu, sigma, W1, b1, g1, be1, W2, b2, g2, be2,
           W3, b3, g3, be3, Wf1, bf1, Wf2, bf2):
    CT = jnp.swapaxes(C, 1, 2)
    nv2 = -_LOG2E / (2.0 * sigma * sigma + 1e-6)

    Vb = jnp.pad(V, ((0, 0), (0, 0), (0, D - V.shape[-1]))).astype(
        jnp.bfloat16)
    W1p = jnp.pad(W1, ((0, 0), (0, D - W1.shape[1]), (0, 0)))
    Wall = jnp.stack([W1p, W2, W3]).astype(jnp.bfloat16)
    ball = jnp.stack([b1, b2, b3])[:, :, None, :]
    gall = jnp.stack([g1, g2, g3])
    beall = jnp.stack([be1, be2, be3])

    Vx = _gcn_call(C, CT, Vb, Wall, ball, gall, beall, mu, nv2)

    W1r = Wf1.reshape(N // POOL, D, FC1)
    out = _head_call(Vx, W1r, bf1[None, :], Wf2, bf2[None, :])
    return out


# speedup vs baseline: 2.2172x; 1.1588x over previous
import jax
import jax.numpy as jnp
from jax.experimental import pallas as pl
from jax.experimental.pallas import tpu as pltpu

B, N, CDIM = 8, 512, 3
K = 10
D = 128
POOL = 4
FC1, FC2 = 512, 128

_LOG2E = 1.4426950408889634


def _normalize_prev(l, b, h_scr, st_scr, gall_ref, beall_ref):
    slot = (l - 1) % 2
    full = st_scr[...]
    st2 = jnp.where(slot == 0, full[0], full[1])
    inv_n = 1.0 / float(B * N)
    mean = st2[0] * inv_n
    var = st2[1] * inv_n - mean * mean
    rstd = jax.lax.rsqrt(var + 1e-5)
    g = gall_ref[l - 1]
    be = beall_ref[l - 1]
    scale = (g * rstd).astype(jnp.bfloat16)
    shift = (be - mean * g * rstd).astype(jnp.bfloat16)
    acc = None
    for kk in range(K):
        hk = h_scr[b * K + kk]
        hn = (hk * scale[kk:kk + 1, :]
              + shift[kk:kk + 1, :]).astype(jnp.float32)
        hs = hn * pl.reciprocal(1.0 + jnp.abs(hn), approx=True)
        acc = hs if acc is None else acc + hs
    return (acc * (1.0 / K)).astype(jnp.bfloat16)


KG = 10
NJ = K // KG


def _gcn_kernel(c_ref, ct_ref, v_ref, wall_ref, ball_ref, gall_ref,
                beall_ref, mu_ref, nv_ref, o_ref,
                a_scr, h_scr, vx_scr, dm_scr, st_scr):
    l = pl.program_id(0)
    b = pl.program_id(1)
    j = pl.program_id(2)

    @pl.when(jnp.logical_and(l == 0, j == 0))
    def _():
        cb = c_ref[0]
        ct = ct_ref[0]
        acc = None
        for cc in range(CDIM):
            dif = cb[:, cc:cc + 1] - ct[cc:cc + 1, :]
            sq = dif * dif
            acc = sq if acc is None else acc + sq
        dm_scr[...] = jnp.sqrt(acc + 1e-12)
        vx_scr[b, :, 0:D] = v_ref[0]
        vx_scr[b, :, D:2 * D] = jnp.ones((N, D), jnp.bfloat16)

    @pl.when(l == 0)
    def _():
        dm = dm_scr[...]
        for kk in range(KG):
            k = j * KG + kk
            d0 = dm - mu_ref[k]
            e = jax.lax.exp2(d0 * d0 * nv_ref[k])
            a_scr[b * K + k] = e.astype(jnp.bfloat16)

    @pl.when(jnp.logical_and(jnp.logical_and(l >= 1, l <= 2), j == 0))
    def _():
        vx_scr[b, :, 0:D] = _normalize_prev(l, b, h_scr, st_scr,
                                            gall_ref, beall_ref)

    @pl.when(jnp.logical_and(l == 3, j == 0))
    def _():
        o_ref[0] = _normalize_prev(l, b, h_scr, st_scr,
                                   gall_ref, beall_ref)

    @pl.when(l <= 2)
    def _():
        lw = jnp.minimum(l, 2)
        ag = a_scr[pl.ds(b * K + j * KG, KG)]
        agr = ag.reshape(KG * N, N)
        maug = jnp.dot(agr, vx_scr[b],
                       preferred_element_type=jnp.float32)
        rs = maug[:, D:D + 1]
        m = maug[:, 0:D] * pl.reciprocal(rs + 1e-9, approx=True)
        m_bf = m.astype(jnp.bfloat16)
        s1l, s2l = [], []
        for kk in range(KG):
            k = j * KG + kk
            h = jnp.dot(m_bf[kk * N:(kk + 1) * N], wall_ref[lw, k],
                        preferred_element_type=jnp.float32) \
                + ball_ref[lw, k]
            h_scr[b * K + k] = h.astype(jnp.bfloat16)
            s1l.append(jnp.sum(h, axis=0, keepdims=True))
            s2l.append(jnp.sum(h * h, axis=0, keepdims=True))
        s1g = jnp.concatenate(s1l, axis=0)
        s2g = jnp.concatenate(s2l, axis=0)
        if KG == K:
            s1k, s2k = s1g, s2g
        else:
            z = jnp.zeros((K - KG, D), jnp.float32)
            s1k = jnp.where(j == 0, jnp.concatenate([s1g, z]),
                            jnp.concatenate([z, s1g]))
            s2k = jnp.where(j == 0, jnp.concatenate([s2g, z]),
                            jnp.concatenate([z, s2g]))
        contrib = jnp.stack([s1k, s2k], axis=0)
        slot = l % 2
        slotmask = (jax.lax.broadcasted_iota(jnp.int32, (2, 2, K, D), 0)
                    == slot)
        first = jnp.logical_and(b == 0, j == 0)
        full = st_scr[...]
        cleared = jnp.where(jnp.logical_and(first, slotmask), 0.0, full)
        st_scr[...] = cleared + jnp.where(slotmask, contrib[None], 0.0)


def _gcn_call(C, CT, Vb, Wall, ball, gall, beall, mu, nv2):
    return pl.pallas_call(
        _gcn_kernel,
        grid=(4, B, NJ),
        in_specs=[
            pl.BlockSpec((1, N, CDIM), lambda l, b, k: (b, 0, 0)),
            pl.BlockSpec((1, CDIM, N), lambda l, b, k: (b, 0, 0)),
            pl.BlockSpec((1, N, D), lambda l, b, k: (b, 0, 0)),
            pl.BlockSpec((3, K, D, D), lambda l, b, k: (0, 0, 0, 0)),
            pl.BlockSpec((3, K, 1, D), lambda l, b, k: (0, 0, 0, 0)),
            pl.BlockSpec((3, K, D), lambda l, b, k: (0, 0, 0)),
            pl.BlockSpec((3, K, D), lambda l, b, k: (0, 0, 0)),
            pl.BlockSpec(memory_space=pltpu.SMEM),
            pl.BlockSpec(memory_space=pltpu.SMEM),
        ],
        out_specs=pl.BlockSpec((1, N, D), lambda l, b, k: (b, 0, 0)),
        out_shape=jax.ShapeDtypeStruct((B, N, D), jnp.bfloat16),
        scratch_shapes=[
            pltpu.VMEM((B * K, N, N), jnp.bfloat16),
            pltpu.VMEM((B * K, N, D), jnp.bfloat16),
            pltpu.VMEM((B, N, 2 * D), jnp.bfloat16),
            pltpu.VMEM((N, N), jnp.float32),
            pltpu.VMEM((2, 2, K, D), jnp.float32),
        ],
        compiler_params=pltpu.CompilerParams(
            dimension_semantics=("arbitrary", "arbitrary", "arbitrary")),
    )(C, CT, Vb, Wall, ball, gall, beall, mu, nv2)


def _head_kernel(vx_ref, w1_ref, bf1_ref, w2_ref, bf2_ref, o_ref, acc_ref):
    j = pl.program_id(0)
    nj = pl.num_programs(0)
    d = vx_ref.shape[-1]
    rows = w1_ref.shape[0]

    @pl.when(j == 0)
    def _():
        acc_ref[...] = jnp.zeros_like(acc_ref)

    v = vx_ref[...]
    p = v.reshape(B, rows, POOL, d).max(axis=2)
    part = None
    for i in range(rows):
        t = jnp.dot(p[:, i, :], w1_ref[i].astype(jnp.bfloat16),
                    preferred_element_type=jnp.float32)
        part = t if part is None else part + t
    acc_ref[...] += part

    @pl.when(j == nj - 1)
    def _():
        h1 = jnp.maximum(acc_ref[...] + bf1_ref[...], 0.0)
        o = jnp.dot(h1.astype(jnp.bfloat16),
                    w2_ref[...].astype(jnp.bfloat16),
                    preferred_element_type=jnp.float32) + bf2_ref[...]
        o_ref[...] = jnp.maximum(o, 0.0)


def _head_call(Vx, W1r, bf1, W2, bf2):
    d = Vx.shape[-1]
    n2 = N // POOL
    rows = 16
    nsteps = n2 // rows
    return pl.pallas_call(
        _head_kernel,
        grid=(nsteps,),
        in_specs=[
            pl.BlockSpec((B, rows * POOL, d), lambda j: (0, j, 0)),
            pl.BlockSpec((rows, d, FC1), lambda j: (j, 0, 0)),
            pl.BlockSpec((1, FC1), lambda j: (0, 0)),
            pl.BlockSpec((FC1, FC2), lambda j: (0, 0)),
            pl.BlockSpec((1, FC2), lambda j: (0, 0)),
        ],
        out_specs=pl.BlockSpec((B, FC2), lambda j: (0, 0)),
        out_shape=jax.ShapeDtypeStruct((B, FC2), jnp.float32),
        scratch_shapes=[pltpu.VMEM((B, FC1), jnp.float32)],
        compiler_params=pltpu.CompilerParams(
            dimension_semantics=("arbitrary",)),
    )(Vx, W1r, bf1, W2, bf2)


def kernel(V, C, mu, sigma, W1, b1, g1, be1, W2, b2, g2, be2,
           W3, b3, g3, be3, Wf1, bf1, Wf2, bf2):
    CT = jnp.swapaxes(C, 1, 2)
    nv2 = -_LOG2E / (2.0 * sigma * sigma + 1e-6)

    Vb = jnp.pad(V, ((0, 0), (0, 0), (0, D - V.shape[-1]))).astype(
        jnp.bfloat16)
    W1p = jnp.pad(W1, ((0, 0), (0, D - W1.shape[1]), (0, 0)))
    Wall = jnp.stack([W1p, W2, W3]).astype(jnp.bfloat16)
    ball = jnp.stack([b1, b2, b3])[:, :, None, :]
    gall = jnp.stack([g1, g2, g3])
    beall = jnp.stack([be1, be2, be3])

    Vx = _gcn_call(C, CT, Vb, Wall, ball, gall, beall, mu, nv2)

    W1r = Wf1.reshape(N // POOL, D, FC1)
    out = _head_call(Vx, W1r, bf1[None, :], Wf2, bf2[None, :])
    return out


# speedup vs baseline: 2.2325x; 1.0069x over previous
import jax
import jax.numpy as jnp
from jax.experimental import pallas as pl
from jax.experimental.pallas import tpu as pltpu

B, N, CDIM = 8, 512, 3
K = 10
D = 128
POOL = 4
FC1, FC2 = 512, 128

_LOG2E = 1.4426950408889634


def _normalize_prev(l, b, h_scr, st_scr, gall_ref, beall_ref):
    slot = (l - 1) % 2
    full = st_scr[...]
    st2 = jnp.where(slot == 0, full[0], full[1])
    inv_n = 1.0 / float(B * N)
    mean = st2[0] * inv_n
    var = st2[1] * inv_n - mean * mean
    rstd = jax.lax.rsqrt(var + 1e-5)
    g = gall_ref[l - 1]
    be = beall_ref[l - 1]
    scale = (g * rstd).astype(jnp.bfloat16)
    shift = (be - mean * g * rstd).astype(jnp.bfloat16)
    acc = None
    for kk in range(K):
        hk = h_scr[b * K + kk]
        hn = (hk * scale[kk:kk + 1, :]
              + shift[kk:kk + 1, :]).astype(jnp.float32)
        hs = hn * pl.reciprocal(1.0 + jnp.abs(hn), approx=True)
        acc = hs if acc is None else acc + hs
    return (acc * (1.0 / K)).astype(jnp.bfloat16)


KG = 10
NJ = K // KG


def _gcn_kernel(c_ref, ct_ref, v_ref, wp_ref, ball_ref, gall_ref,
                beall_ref, mu_ref, nv_ref, o_ref,
                a_scr, h_scr, vx_scr, st_scr):
    l = pl.program_id(0)
    b = pl.program_id(1)
    j = pl.program_id(2)

    @pl.when(l == 0)
    def _():
        vx_scr[b, :, 0:D] = v_ref[0]
        vx_scr[b, :, D:2 * D] = jnp.ones((N, D), jnp.bfloat16)
        cb = c_ref[0]
        ct = ct_ref[0]
        acc = None
        for cc in range(CDIM):
            dif = cb[:, cc:cc + 1] - ct[cc:cc + 1, :]
            sq = dif * dif
            acc = sq if acc is None else acc + sq
        dm = jnp.sqrt(acc + 1e-12)
        for kk in range(K):
            d0 = dm - mu_ref[kk]
            e = jax.lax.exp2(d0 * d0 * nv_ref[kk])
            a_scr[b * K + kk] = e.astype(jnp.bfloat16)

    @pl.when(jnp.logical_and(jnp.logical_and(l >= 1, l <= 2), j == 0))
    def _():
        vx_scr[b, :, 0:D] = _normalize_prev(l, b, h_scr, st_scr,
                                            gall_ref, beall_ref)

    @pl.when(jnp.logical_and(l == 3, j == 0))
    def _():
        o_ref[0] = _normalize_prev(l, b, h_scr, st_scr,
                                   gall_ref, beall_ref)

    @pl.when(l <= 2)
    def _():
        lw = jnp.minimum(l, 2)
        ag = a_scr[pl.ds(b * K + j * KG, KG)]
        agr = ag.reshape(KG * N, N)
        maug = jnp.dot(agr, vx_scr[b],
                       preferred_element_type=jnp.float32)
        rs = maug[:, D:D + 1]
        m = maug[:, 0:D] * pl.reciprocal(rs + 1e-9, approx=True)
        m_bf = m.astype(jnp.bfloat16)
        s1l, s2l = [], []
        for pp in range(KG // 2):
            kk = 2 * pp
            k = j * KG + kk
            mcat = jnp.concatenate(
                [m_bf[kk * N:(kk + 1) * N],
                 m_bf[(kk + 1) * N:(kk + 2) * N]], axis=1)
            h2 = jnp.dot(mcat, wp_ref[lw, pp],
                         preferred_element_type=jnp.float32)
            for q in range(2):
                h = h2[:, q * D:(q + 1) * D] + ball_ref[lw, k + q]
                h_scr[b * K + k + q] = h.astype(jnp.bfloat16)
                s1l.append(jnp.sum(h, axis=0, keepdims=True))
                s2l.append(jnp.sum(h * h, axis=0, keepdims=True))
        s1g = jnp.concatenate(s1l, axis=0)
        s2g = jnp.concatenate(s2l, axis=0)
        if KG == K:
            s1k, s2k = s1g, s2g
        else:
            z = jnp.zeros((K - KG, D), jnp.float32)
            s1k = jnp.where(j == 0, jnp.concatenate([s1g, z]),
                            jnp.concatenate([z, s1g]))
            s2k = jnp.where(j == 0, jnp.concatenate([s2g, z]),
                            jnp.concatenate([z, s2g]))
        contrib = jnp.stack([s1k, s2k], axis=0)
        slot = l % 2
        slotmask = (jax.lax.broadcasted_iota(jnp.int32, (2, 2, K, D), 0)
                    == slot)
        first = jnp.logical_and(b == 0, j == 0)
        full = st_scr[...]
        cleared = jnp.where(jnp.logical_and(first, slotmask), 0.0, full)
        st_scr[...] = cleared + jnp.where(slotmask, contrib[None], 0.0)


def _gcn_call(C, CT, Vb, Wpair, ball, gall, beall, mu, nv2):
    return pl.pallas_call(
        _gcn_kernel,
        grid=(4, B, NJ),
        in_specs=[
            pl.BlockSpec((1, N, CDIM), lambda l, b, k: (b, 0, 0)),
            pl.BlockSpec((1, CDIM, N), lambda l, b, k: (b, 0, 0)),
            pl.BlockSpec((1, N, D), lambda l, b, k: (b, 0, 0)),
            pl.BlockSpec((3, K // 2, 2 * D, 2 * D),
                         lambda l, b, k: (0, 0, 0, 0)),
            pl.BlockSpec((3, K, 1, D), lambda l, b, k: (0, 0, 0, 0)),
            pl.BlockSpec((3, K, D), lambda l, b, k: (0, 0, 0)),
            pl.BlockSpec((3, K, D), lambda l, b, k: (0, 0, 0)),
            pl.BlockSpec(memory_space=pltpu.SMEM),
            pl.BlockSpec(memory_space=pltpu.SMEM),
        ],
        out_specs=pl.BlockSpec((1, N, D), lambda l, b, k: (b, 0, 0)),
        out_shape=jax.ShapeDtypeStruct((B, N, D), jnp.bfloat16),
        scratch_shapes=[
            pltpu.VMEM((B * K, N, N), jnp.bfloat16),
            pltpu.VMEM((B * K, N, D), jnp.bfloat16),
            pltpu.VMEM((B, N, 2 * D), jnp.bfloat16),
            pltpu.VMEM((2, 2, K, D), jnp.float32),
        ],
        compiler_params=pltpu.CompilerParams(
            dimension_semantics=("arbitrary", "arbitrary", "arbitrary")),
    )(C, CT, Vb, Wpair, ball, gall, beall, mu, nv2)


def _head_kernel(vx_ref, w1_ref, bf1_ref, w2_ref, bf2_ref, o_ref, acc_ref):
    j = pl.program_id(0)
    nj = pl.num_programs(0)
    d = vx_ref.shape[-1]
    rows = w1_ref.shape[0]

    @pl.when(j == 0)
    def _():
        acc_ref[...] = jnp.zeros_like(acc_ref)

    v = vx_ref[...]
    p = v.reshape(B, rows, POOL, d).max(axis=2)
    part = None
    for i in range(rows):
        t = jnp.dot(p[:, i, :], w1_ref[i].astype(jnp.bfloat16),
                    preferred_element_type=jnp.float32)
        part = t if part is None else part + t
    acc_ref[...] += part

    @pl.when(j == nj - 1)
    def _():
        h1 = jnp.maximum(acc_ref[...] + bf1_ref[...], 0.0)
        o = jnp.dot(h1.astype(jnp.bfloat16),
                    w2_ref[...].astype(jnp.bfloat16),
                    preferred_element_type=jnp.float32) + bf2_ref[...]
        o_ref[...] = jnp.maximum(o, 0.0)


def _head_call(Vx, W1r, bf1, W2, bf2):
    d = Vx.shape[-1]
    n2 = N // POOL
    rows = 16
    nsteps = n2 // rows
    return pl.pallas_call(
        _head_kernel,
        grid=(nsteps,),
        in_specs=[
            pl.BlockSpec((B, rows * POOL, d), lambda j: (0, j, 0)),
            pl.BlockSpec((rows, d, FC1), lambda j: (j, 0, 0)),
            pl.BlockSpec((1, FC1), lambda j: (0, 0)),
            pl.BlockSpec((FC1, FC2), lambda j: (0, 0)),
            pl.BlockSpec((1, FC2), lambda j: (0, 0)),
        ],
        out_specs=pl.BlockSpec((B, FC2), lambda j: (0, 0)),
        out_shape=jax.ShapeDtypeStruct((B, FC2), jnp.float32),
        scratch_shapes=[pltpu.VMEM((B, FC1), jnp.float32)],
        compiler_params=pltpu.CompilerParams(
            dimension_semantics=("arbitrary",)),
    )(Vx, W1r, bf1, W2, bf2)


def kernel(V, C, mu, sigma, W1, b1, g1, be1, W2, b2, g2, be2,
           W3, b3, g3, be3, Wf1, bf1, Wf2, bf2):
    CT = jnp.swapaxes(C, 1, 2)
    nv2 = -_LOG2E / (2.0 * sigma * sigma + 1e-6)

    Vb = jnp.pad(V, ((0, 0), (0, 0), (0, D - V.shape[-1]))).astype(
        jnp.bfloat16)
    W1p = jnp.pad(W1, ((0, 0), (0, D - W1.shape[1]), (0, 0)))
    Wl = jnp.stack([W1p, W2, W3])
    Wr = Wl.reshape(3, K // 2, 2, D, D)
    zpad = jnp.zeros((3, K // 2, D, D), Wl.dtype)
    wtop = jnp.concatenate([Wr[:, :, 0], zpad], axis=-1)
    wbot = jnp.concatenate([zpad, Wr[:, :, 1]], axis=-1)
    Wpair = jnp.concatenate([wtop, wbot], axis=2).astype(jnp.bfloat16)
    ball = jnp.stack([b1, b2, b3])[:, :, None, :]
    gall = jnp.stack([g1, g2, g3])
    beall = jnp.stack([be1, be2, be3])

    Vx = _gcn_call(C, CT, Vb, Wpair, ball, gall, beall, mu, nv2)

    W1r = Wf1.reshape(N // POOL, D, FC1)
    out = _head_call(Vx, W1r, bf1[None, :], Wf2, bf2[None, :])
    return out
